# baseline (speedup 1.0000x reference)
import functools

import jax
import jax.numpy as jnp
from jax import lax
from jax.experimental import pallas as pl
from jax.experimental.pallas import tpu as pltpu
from jax.experimental.pallas import tpu_sc as plsc

_NC = 2
_NS = 16
_NW = _NC * _NS

_BATCH = 4096
_SEQ = 50
_XPAD = 128
_EMB = 64
_NV = _EMB // 16
_BPW = _BATCH // _NW
_RPC = 2
_SEQ8 = 56
_IPC = _RPC * _SEQ8
_NCHUNK = _BPW // _RPC
_NBUF = 4


def _sc_pool(x_hbm, table_hbm, out_hbm, idx_v, b0, b1, b2, b3, out_v, s0, s1, s2, s3):
    bufs = (b0, b1, b2, b3)
    sems = (s0, s1, s2, s3)
    wid = lax.axis_index("s") * _NC + lax.axis_index("c")

    pltpu.sync_copy(x_hbm.at[pl.ds(wid * _BPW, _BPW)], idx_v)

    def start(c, u):
        pltpu.make_async_copy(
            table_hbm.at[idx_v.at[2 * c, pl.ds(0, _SEQ8)]],
            bufs[u].at[pl.ds(0, _SEQ8)], sems[u],
        ).start()
        pltpu.make_async_copy(
            table_hbm.at[idx_v.at[2 * c + 1, pl.ds(0, _SEQ8)]],
            bufs[u].at[pl.ds(_SEQ8, _SEQ8)], sems[u],
        ).start()

    def wait(u):
        pltpu.make_async_copy(
            table_hbm.at[idx_v.at[0, pl.ds(0, _SEQ8)]], bufs[u], sems[u]
        ).wait()

    def process(c, u):
        buf = bufs[u]
        zeros = jnp.zeros((16,), jnp.float32)

        def red_body(s, carry):
            a = list(carry)
            for k in range(2):
                for r in range(_RPC):
                    for d in range(_NV):
                        a[r * _NV + d] = a[r * _NV + d] + buf[
                            r * _SEQ8 + 2 * s + k, pl.ds(d * 16, 16)
                        ]
            return tuple(a)

        acc = lax.fori_loop(0, _SEQ // 2, red_body, (zeros,) * (_RPC * _NV))
        for r in range(_RPC):
            for d in range(_NV):
                out_v[c, pl.ds(r * _EMB + d * 16, 16)] = acc[r * _NV + d]

    for u in range(_NBUF):
        start(u, u)

    def outer(g, _):
        for u in range(_NBUF):
            wait(u)
            process(_NBUF * g + u, u)
            start(_NBUF * g + u + _NBUF, u)
        return 0

    lax.fori_loop(0, _NCHUNK // _NBUF - 1, outer, 0)
    for u in range(_NBUF):
        wait(u)
        process(_NCHUNK - _NBUF + u, u)

    pltpu.sync_copy(out_v, out_hbm.at[pl.ds(wid * (_BPW // 2), _BPW // 2)])


def _pooled_sum(x_pad, table):
    mesh = plsc.VectorSubcoreMesh(core_axis_name="c", subcore_axis_name="s")
    return pl.kernel(
        _sc_pool,
        mesh=mesh,
        compiler_params=pltpu.CompilerParams(use_tc_tiling_on_sc=False),
        out_type=jax.ShapeDtypeStruct((_BATCH // 2, 2 * _EMB), jnp.float32),
        scratch_types=[
            pltpu.VMEM((_BPW, _XPAD), jnp.int32),
            pltpu.VMEM((_IPC, _EMB), jnp.float32),
            pltpu.VMEM((_IPC, _EMB), jnp.float32),
            pltpu.VMEM((_IPC, _EMB), jnp.float32),
            pltpu.VMEM((_IPC, _EMB), jnp.float32),
            pltpu.VMEM((_BPW // 2, 2 * _EMB), jnp.float32),
            pltpu.SemaphoreType.DMA,
            pltpu.SemaphoreType.DMA,
            pltpu.SemaphoreType.DMA,
            pltpu.SemaphoreType.DMA,
        ],
    )(x_pad, table)


def _mlp_body(h_ref, w1t_ref, b1_ref, w2t_ref, b2_ref, o_ref):
    h2 = h_ref[...] * (1.0 / _SEQ)
    h = jnp.concatenate([h2[:, :_EMB], h2[:, _EMB:]], axis=0)
    a = jnp.dot(h, w1t_ref[...], preferred_element_type=jnp.float32) + b1_ref[...]
    a = jnp.maximum(a, 0.0)
    z = jnp.dot(a, w2t_ref[...], preferred_element_type=jnp.float32) + b2_ref[...]
    z = jax.nn.sigmoid(z)
    o_ref[...] = jnp.concatenate([z[: _BATCH // 2], z[_BATCH // 2 :]], axis=1)


def kernel(x, table, W1, b1, W2, b2):
    x_pad = jnp.pad(x.astype(jnp.int32), ((0, 0), (0, _XPAD - _SEQ)))
    h_sum = _pooled_sum(x_pad, table)

    out = pl.pallas_call(
        _mlp_body,
        out_shape=jax.ShapeDtypeStruct((_BATCH // 2, 2), jnp.float32),
    )(h_sum, W1.T, b1.reshape(1, 32), W2.T, b2.reshape(1, 1))
    return out.reshape(_BATCH)


# speedup vs baseline: 5.2619x; 5.2619x over previous
import functools

import jax
import jax.numpy as jnp
from jax import lax
from jax.experimental import pallas as pl
from jax.experimental.pallas import tpu as pltpu
from jax.experimental.pallas import tpu_sc as plsc

_NC = 2
_NS = 16
_NW = _NC * _NS

_BATCH = 4096
_SEQ = 50
_EMB = 64
_NV = _EMB // 16
_BPW = _BATCH // _NW
_RPC = 2
_IPC = _RPC * _SEQ
_NCHUNK = _BPW // _RPC
_NBUF = 4


def _sc_pool(x_hbm, table_hbm, out_hbm, xf_v, idx_v, b0, b1, b2, b3, out_v,
             s0, s1, s2, s3):
    bufs = (b0, b1, b2, b3)
    sems = (s0, s1, s2, s3)
    wid = lax.axis_index("s") * _NC + lax.axis_index("c")

    pltpu.sync_copy(x_hbm.at[pl.ds(wid * _BPW, _BPW)], xf_v)

    def conv_row(r, _):
        for o in (0, 16, 32, 34):
            idx_v[r, pl.ds(o, 16)] = xf_v[r, pl.ds(o, 16)].astype(jnp.int32)
        return 0

    lax.fori_loop(0, _BPW, conv_row, 0)

    def start(c, u):
        pltpu.make_async_copy(
            table_hbm.at[idx_v.at[2 * c]], bufs[u].at[pl.ds(0, _SEQ)], sems[u]
        ).start()
        pltpu.make_async_copy(
            table_hbm.at[idx_v.at[2 * c + 1]], bufs[u].at[pl.ds(_SEQ, _SEQ)],
            sems[u],
        ).start()

    def wait(u):
        pltpu.make_async_copy(table_hbm.at[idx_v.at[0]], bufs[u], sems[u]).wait()

    def process(c, u):
        buf = bufs[u]
        zeros = jnp.zeros((16,), jnp.float32)

        def red_body(s, carry):
            a = list(carry)
            for k in range(2):
                for r in range(_RPC):
                    for d in range(_NV):
                        a[r * _NV + d] = a[r * _NV + d] + buf[
                            r * _SEQ + 2 * s + k, pl.ds(d * 16, 16)
                        ]
            return tuple(a)

        acc = lax.fori_loop(0, _SEQ // 2, red_body, (zeros,) * (_RPC * _NV))
        for r in range(_RPC):
            for d in range(_NV):
                out_v[c, pl.ds(r * _EMB + d * 16, 16)] = acc[r * _NV + d]

    for u in range(_NBUF):
        start(u, u)

    def outer(g, _):
        for u in range(_NBUF):
            wait(u)
            process(_NBUF * g + u, u)
            start(_NBUF * g + u + _NBUF, u)
        return 0

    lax.fori_loop(0, _NCHUNK // _NBUF - 1, outer, 0)
    for u in range(_NBUF):
        wait(u)
        process(_NCHUNK - _NBUF + u, u)

    pltpu.sync_copy(out_v, out_hbm.at[pl.ds(wid * (_BPW // 2), _BPW // 2)])


def _pooled_sum(x_f, table):
    mesh = plsc.VectorSubcoreMesh(core_axis_name="c", subcore_axis_name="s")
    return pl.kernel(
        _sc_pool,
        mesh=mesh,
        compiler_params=pltpu.CompilerParams(use_tc_tiling_on_sc=False),
        out_type=jax.ShapeDtypeStruct((_BATCH // 2, 2 * _EMB), jnp.float32),
        scratch_types=[
            pltpu.VMEM((_BPW, _SEQ), jnp.float32),
            pltpu.VMEM((_BPW, _SEQ), jnp.int32),
            pltpu.VMEM((_IPC, _EMB), jnp.float32),
            pltpu.VMEM((_IPC, _EMB), jnp.float32),
            pltpu.VMEM((_IPC, _EMB), jnp.float32),
            pltpu.VMEM((_IPC, _EMB), jnp.float32),
            pltpu.VMEM((_BPW // 2, 2 * _EMB), jnp.float32),
            pltpu.SemaphoreType.DMA,
            pltpu.SemaphoreType.DMA,
            pltpu.SemaphoreType.DMA,
            pltpu.SemaphoreType.DMA,
        ],
    )(x_f, table)


def _mlp_body(h_ref, w1t_ref, b1_ref, w2t_ref, b2_ref, o_ref):
    h2 = h_ref[...] * (1.0 / _SEQ)
    h = jnp.concatenate([h2[:, :_EMB], h2[:, _EMB:]], axis=0)
    a = jnp.dot(h, w1t_ref[...], preferred_element_type=jnp.float32) + b1_ref[...]
    a = jnp.maximum(a, 0.0)
    z = jnp.dot(a, w2t_ref[...], preferred_element_type=jnp.float32) + b2_ref[...]
    z = jax.nn.sigmoid(z)
    o_ref[...] = jnp.concatenate([z[: _BATCH // 2], z[_BATCH // 2 :]], axis=1)


def kernel(x, table, W1, b1, W2, b2):
    x_f = x.astype(jnp.float32)
    h_sum = _pooled_sum(x_f, table)

    out = pl.pallas_call(
        _mlp_body,
        out_shape=jax.ShapeDtypeStruct((_BATCH // 2, 2), jnp.float32),
    )(h_sum, W1.T, b1.reshape(1, 32), W2.T, b2.reshape(1, 1))
    return out.reshape(_BATCH)
